# baseline (speedup 1.0000x reference)
import functools

import jax
import jax.numpy as jnp
from jax import lax
from jax.experimental import pallas as pl
from jax.experimental.pallas import tpu as pltpu
from jax.experimental.pallas import tpu_sc as plsc

MIN_TEMPERATURE = 1e-05
NEG_BIG = -1e30

_NC = 2
_NS = 16
_NW = _NC * _NS

_CHUNK = 80
_GROUP = 5


def _node_h_kernel(nt_ref, h_ref, out_ref):
    out_ref[...] = lax.dot_general(
        nt_ref[...], h_ref[...],
        (((1,), (1,)), ((), ())),
        preferred_element_type=jnp.float32,
    )


def _make_sc_gather(n_flat, num_edges, chunk=_CHUNK, group=_GROUP):
    ew = num_edges // _NW
    assert ew * _NW == num_edges and ew % chunk == 0
    n_chunks = ew // chunk
    assert n_chunks % group == 0 and chunk % 8 == 0 and chunk <= 128
    mesh = plsc.VectorSubcoreMesh(
        core_axis_name="c", subcore_axis_name="s",
        num_cores=_NC, num_subcores=_NS,
    )

    @functools.partial(
        pl.kernel,
        out_type=jax.ShapeDtypeStruct((num_edges,), jnp.float32),
        mesh=mesh,
        scratch_types=[
            pltpu.VMEM((ew,), jnp.int32),
            pltpu.VMEM((ew,), jnp.int32),
            pltpu.VMEM((ew,), jnp.int32),
            pltpu.VMEM((ew,), jnp.float32),
            pltpu.SemaphoreType.DMA,
        ],
    )
    def sc_gather(table_hbm, dst_hbm, gid_hbm, out_hbm,
                  dstv, gidv, idxv, gv, sem):
        wid = lax.axis_index("s") * _NC + lax.axis_index("c")
        base = wid * ew
        pltpu.sync_copy(dst_hbm.at[pl.ds(base, ew)], dstv)
        pltpu.sync_copy(gid_hbm.at[pl.ds(base, ew)], gidv)

        def idx_body(j, carry):
            for k in range(5):
                sl = pl.ds(j * 80 + k * 16, 16)
                idxv[sl] = dstv[sl] * 64 + gidv[sl]
            return carry

        lax.fori_loop(0, ew // 80, idx_body, 0, unroll=False)

        def gather_body(j, carry):
            copies = []
            for k in range(group):
                off = (j * group + k) * chunk
                copies.append(pltpu.async_copy(
                    table_hbm.at[idxv.at[pl.ds(off, chunk)]],
                    gv.at[pl.ds(off, chunk)],
                    sem,
                ))
            for cp in copies:
                cp.wait()
            return carry

        lax.fori_loop(0, n_chunks // group, gather_body, 0, unroll=False)
        pltpu.sync_copy(gv, out_hbm.at[pl.ds(base, ew)])

    return sc_gather


def _stream_step(tile_e, ak_ref, g_ref, gid_ref, h_ref, temp_ref,
                 m0_ref, s0_ref, acc_max, acc_sum):
    i = pl.program_id(0)
    b = h_ref.shape[0]
    d = h_ref.shape[1]

    @pl.when(i == 0)
    def _init():
        acc_max[...] = m0_ref[...]
        acc_sum[...] = s0_ref[...]

    temp = jnp.maximum(temp_ref[0, 0], MIN_TEMPERATURE)
    inv_scale = 1.0 / (jnp.sqrt(jnp.float32(d)) * temp)

    scores_t = lax.dot_general(
        h_ref[...], ak_ref[...],
        (((1,), (1,)), ((), ())),
        preferred_element_type=jnp.float32,
    )
    gid_row = gid_ref[0]
    onehot = lax.broadcasted_iota(jnp.int32, (b, tile_e), 0) == gid_row
    oh_f = onehot.astype(jnp.float32)
    sel_t = jnp.sum(oh_f * scores_t, axis=0, keepdims=True)
    es_t = (sel_t + g_ref[0]) * inv_scale

    mt = jnp.max(es_t)
    tile_max = jnp.max(jnp.where(onehot, es_t, NEG_BIG), axis=1,
                       keepdims=True)
    m_old = acc_max[...]
    m_new = jnp.maximum(m_old, tile_max)
    p_t = jnp.exp(es_t - mt)
    q = lax.dot_general(
        oh_f, p_t, (((1,), (1,)), ((), ())),
        preferred_element_type=jnp.float32,
    )
    factor = jnp.exp(jnp.minimum(mt - m_new, 80.0))
    acc_sum[...] = acc_sum[...] * jnp.exp(m_old - m_new) + q * factor
    acc_max[...] = m_new


def _partial_kernel(num_tiles, tile_e, ak_ref, g_ref, gid_ref, h_ref,
                    temp_ref, m0_ref, s0_ref, outm_ref, outs_ref,
                    acc_max, acc_sum):
    _stream_step(tile_e, ak_ref, g_ref, gid_ref, h_ref, temp_ref,
                 m0_ref, s0_ref, acc_max, acc_sum)

    @pl.when(pl.program_id(0) == num_tiles - 1)
    def _emit():
        outm_ref[...] = acc_max[...]
        outs_ref[...] = acc_sum[...]


def _final_kernel(num_tiles, tile_e, ak_ref, g_ref, gid_ref, h_ref,
                  temp_ref, m0_ref, s0_ref, se_ref, lng_ref, lnb_ref,
                  sw_ref, sb_ref, out_ref, acc_max, acc_sum):
    _stream_step(tile_e, ak_ref, g_ref, gid_ref, h_ref, temp_ref,
                 m0_ref, s0_ref, acc_max, acc_sum)
    b = h_ref.shape[0]
    d = h_ref.shape[1]
    temp = jnp.maximum(temp_ref[0, 0], MIN_TEMPERATURE)

    @pl.when(pl.program_id(0) == num_tiles - 1)
    def _final():
        hid = h_ref[...]
        se = se_ref[...]
        n_feat = jnp.float32(d + se.shape[1])
        s1 = jnp.sum(hid, axis=1) + jnp.sum(se, axis=1)
        s2 = jnp.sum(hid * hid, axis=1) + jnp.sum(se * se, axis=1)
        mu = s1 / n_feat
        var = s2 / n_feat - mu * mu
        inv_std = lax.rsqrt(var + 1e-5)
        gw = lng_ref[0, :] * sw_ref[0, :]
        dots = (jnp.sum(hid * gw[:d][None, :], axis=1)
                + jnp.sum(se * gw[d:][None, :], axis=1))
        sum_gw = jnp.sum(gw)
        bw = jnp.sum(lnb_ref[0, :] * sw_ref[0, :])
        stop_logit = inv_std * (dots - mu * sum_gw) + bw + sb_ref[0, 0]
        ss = stop_logit / temp

        seg_max = acc_max[:, 0]
        seg_sum = acc_sum[:, 0]
        m = jnp.maximum(seg_max, ss)
        denom = seg_sum * jnp.exp(seg_max - m) + jnp.exp(ss - m)
        log_denom = jnp.log(denom)
        lp_stop = ss - m - log_denom
        lp_edge = seg_max - m - log_denom
        out_ref[0, :] = jnp.where(ss > seg_max, lp_stop, lp_edge)


def kernel(node_tokens, action_keys, hidden, edge_index, edge_graph_ids,
           stop_extra, ln_gamma, ln_beta, stop_w, stop_b, temperature):
    n, d = node_tokens.shape
    e = action_keys.shape[0]
    b = hidden.shape[0]

    node_h = pl.pallas_call(
        _node_h_kernel,
        out_shape=jax.ShapeDtypeStruct((n, b), jnp.float32),
    )(node_tokens, hidden)

    dst = edge_index[1]
    table = node_h.reshape(-1)
    e1, e2 = 25600, 38400
    e12 = e1 + e2
    e3 = e - e12
    g1 = _make_sc_gather(n * b, e1)(
        table, dst[:e1], edge_graph_ids[:e1])
    g2 = _make_sc_gather(n * b, e2)(
        table, dst[e1:e12], edge_graph_ids[e1:e12])
    g3 = _make_sc_gather(n * b, e3, group=10)(
        table, dst[e12:], edge_graph_ids[e12:])

    temp2 = temperature.reshape(1, 1)
    m0 = jnp.full((b, 1), NEG_BIG, jnp.float32)
    s0 = jnp.zeros((b, 1), jnp.float32)

    def specs(tile, off):
        return [
            pl.BlockSpec((tile, d), lambda i: (i + off, 0)),
            pl.BlockSpec((1, 1, tile), lambda i: (i, 0, 0)),
            pl.BlockSpec((1, 1, tile), lambda i: (i + off, 0, 0)),
            pl.BlockSpec((b, d), lambda i: (0, 0)),
            pl.BlockSpec((1, 1), lambda i: (0, 0)),
            pl.BlockSpec((b, 1), lambda i: (0, 0)),
            pl.BlockSpec((b, 1), lambda i: (0, 0)),
        ]

    acc_scratch = [
        pltpu.VMEM((b, 1), jnp.float32),
        pltpu.VMEM((b, 1), jnp.float32),
    ]
    acc_out_specs = [pl.BlockSpec((b, 1), lambda i: (0, 0))] * 2
    acc_out_shape = [jax.ShapeDtypeStruct((b, 1), jnp.float32)] * 2

    tile_s = 12800
    gid3s = edge_graph_ids[:e12].reshape(e12 // tile_s, 1, tile_s)
    nt1 = e1 // tile_s
    m1, s1 = pl.pallas_call(
        functools.partial(_partial_kernel, nt1, tile_s),
        grid=(nt1,),
        in_specs=specs(tile_s, 0),
        out_specs=acc_out_specs,
        out_shape=acc_out_shape,
        scratch_shapes=acc_scratch,
    )(
        action_keys, g1.reshape(nt1, 1, tile_s), gid3s,
        hidden, temp2, m0, s0,
    )

    nt2 = e2 // tile_s
    m2, s2 = pl.pallas_call(
        functools.partial(_partial_kernel, nt2, tile_s),
        grid=(nt2,),
        in_specs=specs(tile_s, nt1),
        out_specs=acc_out_specs,
        out_shape=acc_out_shape,
        scratch_shapes=acc_scratch,
    )(
        action_keys, g2.reshape(nt2, 1, tile_s), gid3s,
        hidden, temp2, m1, s1,
    )

    tile_l = 32000
    nt3 = e3 // tile_l
    off3 = e12 // tile_l
    assert off3 * tile_l == e12 and nt3 * tile_l == e3
    gid3l = edge_graph_ids.reshape(e // tile_l, 1, tile_l)
    out = pl.pallas_call(
        functools.partial(_final_kernel, nt3, tile_l),
        grid=(nt3,),
        in_specs=specs(tile_l, off3) + [
            pl.BlockSpec((b, 2), lambda i: (0, 0)),
            pl.BlockSpec((1, d + 2), lambda i: (0, 0)),
            pl.BlockSpec((1, d + 2), lambda i: (0, 0)),
            pl.BlockSpec((1, d + 2), lambda i: (0, 0)),
            pl.BlockSpec((1, 1), lambda i: (0, 0)),
        ],
        out_specs=pl.BlockSpec((1, b), lambda i: (0, 0)),
        out_shape=jax.ShapeDtypeStruct((1, b), jnp.float32),
        scratch_shapes=acc_scratch,
    )(
        action_keys, g3.reshape(nt3, 1, tile_l), gid3l,
        hidden, temp2, m2, s2,
        stop_extra,
        ln_gamma.reshape(1, -1), ln_beta.reshape(1, -1),
        stop_w.reshape(1, -1), stop_b.reshape(1, 1),
    )
    return out.reshape(b)


# speedup vs baseline: 1.0648x; 1.0648x over previous
import functools

import jax
import jax.numpy as jnp
from jax import lax
from jax.experimental import pallas as pl
from jax.experimental.pallas import tpu as pltpu
from jax.experimental.pallas import tpu_sc as plsc

MIN_TEMPERATURE = 1e-05
NEG_BIG = -1e30

_NC = 2
_NS = 16
_NW = _NC * _NS

_CHUNK = 80
_GROUP = 5


def _node_h_kernel(nt_ref, h_ref, out_ref):
    out_ref[...] = lax.dot_general(
        nt_ref[...], h_ref[...],
        (((1,), (1,)), ((), ())),
        preferred_element_type=jnp.float32,
    )


def _make_sc_gather(n_flat, num_edges, chunk=_CHUNK, group=_GROUP):
    ew = num_edges // _NW
    assert ew * _NW == num_edges and ew % chunk == 0
    n_chunks = ew // chunk
    assert n_chunks % group == 0 and chunk % 8 == 0 and chunk <= 128
    mesh = plsc.VectorSubcoreMesh(
        core_axis_name="c", subcore_axis_name="s",
        num_cores=_NC, num_subcores=_NS,
    )

    @functools.partial(
        pl.kernel,
        out_type=jax.ShapeDtypeStruct((num_edges,), jnp.float32),
        mesh=mesh,
        scratch_types=[
            pltpu.VMEM((ew,), jnp.int32),
            pltpu.VMEM((ew,), jnp.int32),
            pltpu.VMEM((ew,), jnp.int32),
            pltpu.VMEM((ew,), jnp.float32),
            pltpu.SemaphoreType.DMA,
        ],
    )
    def sc_gather(table_hbm, dst_hbm, gid_hbm, out_hbm,
                  dstv, gidv, idxv, gv, sem):
        wid = lax.axis_index("s") * _NC + lax.axis_index("c")
        base = wid * ew
        pltpu.sync_copy(dst_hbm.at[pl.ds(base, ew)], dstv)
        pltpu.sync_copy(gid_hbm.at[pl.ds(base, ew)], gidv)

        def idx_body(j, carry):
            for k in range(5):
                sl = pl.ds(j * 80 + k * 16, 16)
                idxv[sl] = dstv[sl] * 64 + gidv[sl]
            return carry

        lax.fori_loop(0, ew // 80, idx_body, 0, unroll=False)

        def gather_body(j, carry):
            copies = []
            for k in range(group):
                off = (j * group + k) * chunk
                copies.append(pltpu.async_copy(
                    table_hbm.at[idxv.at[pl.ds(off, chunk)]],
                    gv.at[pl.ds(off, chunk)],
                    sem,
                ))
            for cp in copies:
                cp.wait()
            return carry

        lax.fori_loop(0, n_chunks // group, gather_body, 0, unroll=False)
        pltpu.sync_copy(gv, out_hbm.at[pl.ds(base, ew)])

    return sc_gather


def _stream_step(tile_e, ak_ref, g_ref, gid_ref, h_ref, temp_ref,
                 m0_ref, s0_ref, acc_max, acc_sum):
    i = pl.program_id(0)
    b = h_ref.shape[0]
    d = h_ref.shape[1]

    @pl.when(i == 0)
    def _init():
        acc_max[...] = m0_ref[...]
        acc_sum[...] = s0_ref[...]

    temp = jnp.maximum(temp_ref[0, 0], MIN_TEMPERATURE)
    inv_scale = 1.0 / (jnp.sqrt(jnp.float32(d)) * temp)

    scores_t = lax.dot_general(
        h_ref[...], ak_ref[...],
        (((1,), (1,)), ((), ())),
        preferred_element_type=jnp.float32,
    )
    gid_row = gid_ref[0]
    onehot = lax.broadcasted_iota(jnp.int32, (b, tile_e), 0) == gid_row
    oh_f = onehot.astype(jnp.float32)
    sel_t = jnp.sum(oh_f * scores_t, axis=0, keepdims=True)
    es_t = (sel_t + g_ref[0]) * inv_scale

    mt = jnp.max(es_t)
    tile_max = jnp.max(jnp.where(onehot, es_t, NEG_BIG), axis=1,
                       keepdims=True)
    m_old = acc_max[...]
    m_new = jnp.maximum(m_old, tile_max)
    p_t = jnp.exp(es_t - mt)
    q = lax.dot_general(
        oh_f, p_t, (((1,), (1,)), ((), ())),
        preferred_element_type=jnp.float32,
    )
    factor = jnp.exp(jnp.minimum(mt - m_new, 80.0))
    acc_sum[...] = acc_sum[...] * jnp.exp(m_old - m_new) + q * factor
    acc_max[...] = m_new


def _partial_kernel(num_tiles, tile_e, ak_ref, g_ref, gid_ref, h_ref,
                    temp_ref, m0_ref, s0_ref, outm_ref, outs_ref,
                    acc_max, acc_sum):
    _stream_step(tile_e, ak_ref, g_ref, gid_ref, h_ref, temp_ref,
                 m0_ref, s0_ref, acc_max, acc_sum)

    @pl.when(pl.program_id(0) == num_tiles - 1)
    def _emit():
        outm_ref[...] = acc_max[...]
        outs_ref[...] = acc_sum[...]


def _final_kernel(num_tiles, tile_e, ak_ref, g_ref, gid_ref, h_ref,
                  temp_ref, m0_ref, s0_ref, se_ref, lng_ref, lnb_ref,
                  sw_ref, sb_ref, out_ref, acc_max, acc_sum):
    _stream_step(tile_e, ak_ref, g_ref, gid_ref, h_ref, temp_ref,
                 m0_ref, s0_ref, acc_max, acc_sum)
    b = h_ref.shape[0]
    d = h_ref.shape[1]
    temp = jnp.maximum(temp_ref[0, 0], MIN_TEMPERATURE)

    @pl.when(pl.program_id(0) == num_tiles - 1)
    def _final():
        hid = h_ref[...]
        se = se_ref[...]
        n_feat = jnp.float32(d + se.shape[1])
        s1 = jnp.sum(hid, axis=1) + jnp.sum(se, axis=1)
        s2 = jnp.sum(hid * hid, axis=1) + jnp.sum(se * se, axis=1)
        mu = s1 / n_feat
        var = s2 / n_feat - mu * mu
        inv_std = lax.rsqrt(var + 1e-5)
        gw = lng_ref[0, :] * sw_ref[0, :]
        dots = (jnp.sum(hid * gw[:d][None, :], axis=1)
                + jnp.sum(se * gw[d:][None, :], axis=1))
        sum_gw = jnp.sum(gw)
        bw = jnp.sum(lnb_ref[0, :] * sw_ref[0, :])
        stop_logit = inv_std * (dots - mu * sum_gw) + bw + sb_ref[0, 0]
        ss = stop_logit / temp

        seg_max = acc_max[:, 0]
        seg_sum = acc_sum[:, 0]
        m = jnp.maximum(seg_max, ss)
        denom = seg_sum * jnp.exp(seg_max - m) + jnp.exp(ss - m)
        log_denom = jnp.log(denom)
        lp_stop = ss - m - log_denom
        lp_edge = seg_max - m - log_denom
        out_ref[0, :] = jnp.where(ss > seg_max, lp_stop, lp_edge)


def kernel(node_tokens, action_keys, hidden, edge_index, edge_graph_ids,
           stop_extra, ln_gamma, ln_beta, stop_w, stop_b, temperature):
    n, d = node_tokens.shape
    e = action_keys.shape[0]
    b = hidden.shape[0]

    node_h = pl.pallas_call(
        _node_h_kernel,
        out_shape=jax.ShapeDtypeStruct((n, b), jnp.float32),
    )(node_tokens, hidden)

    dst = edge_index[1]
    table = node_h.reshape(-1)
    tile_e = 32000
    e1 = 2 * tile_e
    e2 = e - e1
    g1 = _make_sc_gather(n * b, e1)(
        table, dst[:e1], edge_graph_ids[:e1])
    g2 = _make_sc_gather(n * b, e2, group=10)(
        table, dst[e1:], edge_graph_ids[e1:])

    temp2 = temperature.reshape(1, 1)
    m0 = jnp.full((b, 1), NEG_BIG, jnp.float32)
    s0 = jnp.zeros((b, 1), jnp.float32)

    def specs(off):
        return [
            pl.BlockSpec((tile_e, d), lambda i: (i + off, 0)),
            pl.BlockSpec((1, 1, tile_e), lambda i: (i, 0, 0)),
            pl.BlockSpec((1, 1, tile_e), lambda i: (i + off, 0, 0)),
            pl.BlockSpec((b, d), lambda i: (0, 0)),
            pl.BlockSpec((1, 1), lambda i: (0, 0)),
            pl.BlockSpec((b, 1), lambda i: (0, 0)),
            pl.BlockSpec((b, 1), lambda i: (0, 0)),
        ]

    acc_scratch = [
        pltpu.VMEM((b, 1), jnp.float32),
        pltpu.VMEM((b, 1), jnp.float32),
    ]

    gid3 = edge_graph_ids.reshape(e // tile_e, 1, tile_e)
    nt1 = e1 // tile_e
    m1, s1 = pl.pallas_call(
        functools.partial(_partial_kernel, nt1, tile_e),
        grid=(nt1,),
        in_specs=specs(0),
        out_specs=[pl.BlockSpec((b, 1), lambda i: (0, 0))] * 2,
        out_shape=[jax.ShapeDtypeStruct((b, 1), jnp.float32)] * 2,
        scratch_shapes=acc_scratch,
    )(
        action_keys, g1.reshape(nt1, 1, tile_e),
        gid3,
        hidden, temp2, m0, s0,
    )

    nt2 = e2 // tile_e
    out = pl.pallas_call(
        functools.partial(_final_kernel, nt2, tile_e),
        grid=(nt2,),
        in_specs=specs(nt1) + [
            pl.BlockSpec((b, 2), lambda i: (0, 0)),
            pl.BlockSpec((1, d + 2), lambda i: (0, 0)),
            pl.BlockSpec((1, d + 2), lambda i: (0, 0)),
            pl.BlockSpec((1, d + 2), lambda i: (0, 0)),
            pl.BlockSpec((1, 1), lambda i: (0, 0)),
        ],
        out_specs=pl.BlockSpec((1, b), lambda i: (0, 0)),
        out_shape=jax.ShapeDtypeStruct((1, b), jnp.float32),
        scratch_shapes=acc_scratch,
    )(
        action_keys, g2.reshape(nt2, 1, tile_e),
        gid3,
        hidden, temp2, m1, s1,
        stop_extra,
        ln_gamma.reshape(1, -1), ln_beta.reshape(1, -1),
        stop_w.reshape(1, -1), stop_b.reshape(1, 1),
    )
    return out.reshape(b)
